# baseline (speedup 1.0000x reference)
import functools

import jax
import jax.numpy as jnp
from jax import lax
from jax.experimental import pallas as pl
from jax.experimental.pallas import tpu as pltpu
from jax.experimental.pallas import tpu_sc as plsc

N_USERS = 25000
N_ITEMS = 25000
N = N_USERS + N_ITEMS
D = 64
N_LAYERS = 3
E = 800000

NC = 2
NS = 16
QN = N // 4
AR = 12560
ZCH = AR // NS
WB = 781
K = 512
PAIRS = 49
CHUNKS = 2 * PAIRS
EPT = K * CHUNKS
E_PAD = NS * EPT


def _layer_body(x_hbm, ed_hbm, y_hbm,
                ebuf0, ebuf1, loc0, loc1, rows0, rows1, acc,
                sem_e0, sem_e1, sem_g0, sem_g1, sem_s0, sem_s1):
    c = lax.axis_index("c")
    s = lax.axis_index("s")
    cbase = s * CHUNKS
    banks = ((ebuf0, loc0, rows0, sem_e0, sem_g0, sem_s0),
             (ebuf1, loc1, rows1, sem_e1, sem_g1, sem_s1))

    for p in range(2):
        base_row = (c * 2 + p) * QN

        def _z(i, _):
            z = jnp.zeros((16,), jnp.float32)
            for d in range(D // 16):
                rows0[i, pl.ds(d * 16, 16)] = z
            return 0
        lax.fori_loop(0, K, _z, 0)
        zbase = s * ZCH
        pltpu.sync_copy(rows0.at[pl.ds(0, K)], acc.at[pl.ds(zbase, K)])
        pltpu.sync_copy(rows0.at[pl.ds(0, ZCH - K)],
                        acc.at[pl.ds(zbase + K, ZCH - K)])
        plsc.subcore_barrier()

        def _fetch(g, bank):
            ebuf, _, _, sem_e, _, _ = bank
            return pltpu.async_copy(ed_hbm.at[g], ebuf, sem_e)

        def _drain_scatter(bank):
            _, loc, rows_v, _, _, sem_s = bank
            pltpu.make_async_copy(rows_v, acc.at[loc], sem_s).wait()

        def _gather(bank):
            ebuf, _, rows_v, _, sem_g, _ = bank
            return pltpu.async_copy(x_hbm.at[ebuf.at[1]], rows_v, sem_g)

        def _process(bank):
            ebuf, loc, rows_v, _, _, sem_s = bank

            def _cl(j, _):
                sl = pl.ds(j * 16, 16)
                r = ebuf[0, sl]
                lv = r - base_row
                ok = (lv >= 0) & (lv < QN)
                loc[sl] = jnp.where(ok, lv, QN)
                vv = plsc.bitcast(ebuf[2, sl], jnp.float32)
                for l in range(16):
                    bv = jnp.broadcast_to(vv[l], (16,))
                    e = j * 16 + l
                    for d in range(D // 16):
                        rsl = pl.ds(d * 16, 16)
                        rows_v[e, rsl] = rows_v[e, rsl] * bv
                return 0
            lax.fori_loop(0, K // 16, _cl, 0)
            pltpu.async_copy(rows_v, acc.at[loc], sem_s, add=True)

        def _pair(j, _):
            descs = []
            for b in range(2):
                bank = banks[b]
                de = _fetch(cbase + 2 * j + b, bank)

                @pl.when(j > 0)
                def _():
                    _drain_scatter(bank)

                de.wait()
                descs.append(_gather(bank))
            for b in range(2):
                descs[b].wait()
                _process(banks[b])
            return 0

        lax.fori_loop(0, PAIRS, _pair, 0)
        _drain_scatter(banks[0])
        _drain_scatter(banks[1])
        plsc.subcore_barrier()

        wb = s * WB
        pltpu.sync_copy(acc.at[pl.ds(wb, WB)],
                        y_hbm.at[pl.ds(base_row + wb, WB)])

        @pl.when(s == 0)
        def _():
            pltpu.sync_copy(acc.at[pl.ds(NS * WB, QN - NS * WB)],
                            y_hbm.at[pl.ds(base_row + NS * WB, QN - NS * WB)])

        plsc.subcore_barrier()


_layer = functools.partial(
    pl.kernel,
    out_type=jax.ShapeDtypeStruct((N, D), jnp.float32),
    mesh=plsc.VectorSubcoreMesh(core_axis_name="c", subcore_axis_name="s"),
    compiler_params=pltpu.CompilerParams(
        use_tc_tiling_on_sc=False, needs_layout_passes=False),
    scratch_types=[
        pltpu.VMEM((3, K), jnp.int32),
        pltpu.VMEM((3, K), jnp.int32),
        pltpu.VMEM((K,), jnp.int32),
        pltpu.VMEM((K,), jnp.int32),
        pltpu.VMEM((K, D), jnp.float32),
        pltpu.VMEM((K, D), jnp.float32),
        pltpu.VMEM_SHARED((AR, D), jnp.float32),
        pltpu.SemaphoreType.DMA,
        pltpu.SemaphoreType.DMA,
        pltpu.SemaphoreType.DMA,
        pltpu.SemaphoreType.DMA,
        pltpu.SemaphoreType.DMA,
        pltpu.SemaphoreType.DMA,
    ],
)(_layer_body)


def _mean_body(x0, x1, x2, x3, o):
    o[...] = (x0[...] + x1[...] + x2[...] + x3[...]) * 0.25


def _mean(x0, x1, x2, x3):
    blk = 400
    grid = N // blk
    spec = pl.BlockSpec((blk, D), lambda i: (i, 0))
    return pl.pallas_call(
        _mean_body,
        grid=(grid,),
        in_specs=[spec] * 4,
        out_specs=spec,
        out_shape=jax.ShapeDtypeStruct((N, D), jnp.float32),
    )(x0, x1, x2, x3)


def kernel(adj_indices, adj_values, user_emb, item_emb):
    row = adj_indices[0].astype(jnp.int32)
    col = adj_indices[1].astype(jnp.int32)
    val = adj_values.astype(jnp.float32)

    pad = E_PAD - E
    row = jnp.concatenate([row, jnp.full((pad,), N, jnp.int32)])
    col = jnp.concatenate([col, jnp.zeros((pad,), jnp.int32)])
    vbits = lax.bitcast_convert_type(
        jnp.concatenate([val, jnp.zeros((pad,), jnp.float32)]), jnp.int32)
    edges = jnp.stack(
        [row.reshape(-1, K), col.reshape(-1, K), vbits.reshape(-1, K)],
        axis=1)

    x0 = jnp.concatenate([user_emb, item_emb], axis=0)
    xs = [x0]
    x = x0
    for _ in range(N_LAYERS):
        x = _layer(x, edges)
        xs.append(x)

    out = _mean(*xs)
    return (out[:N_USERS], out[N_USERS:])
